# baseline (speedup 1.0000x reference)
import functools

import jax
import jax.numpy as jnp
from jax.experimental import pallas as pl
from jax.experimental.pallas import tpu as pltpu

NUM_EXPERTS = 8
D_IN = 2048
D_HID = 8192
D_OUT = 2048
N_TOK = 8192

T = 128
HC = 1024
NH = D_HID // HC
N_PAD = N_TOK + NUM_EXPERTS * T
NT = N_PAD // T


def _mlp_body(te_ref, x_ref, w1_ref, b1_ref, w2_ref, b2_ref, o_ref,
              w1s_ref, w2s_ref):
    h = pl.program_id(0)
    t = pl.program_id(1)
    changed = (t == 0) | (te_ref[t] != te_ref[jnp.maximum(t - 1, 0)])

    @pl.when(changed)
    def _():
        w1s_ref[...] = w1_ref[0].astype(jnp.bfloat16)
        w2s_ref[...] = w2_ref[0].astype(jnp.bfloat16)

    xb = x_ref[...]
    hb = jnp.maximum(
        jnp.dot(xb, w1s_ref[...], preferred_element_type=jnp.float32)
        + b1_ref[0, 0],
        0.0).astype(jnp.bfloat16)
    part = jnp.dot(hb, w2s_ref[...], preferred_element_type=jnp.float32)

    @pl.when(h == 0)
    def _():
        o_ref[0] = (part + b2_ref[0]).astype(jnp.bfloat16)

    @pl.when(h != 0)
    def _():
        o_ref[0] = part.astype(jnp.bfloat16)


@functools.partial(jax.jit, static_argnames=())
def _grouped_mlp(x_pad, tile_expert, W1, b1, W2, b2):
    grid_spec = pltpu.PrefetchScalarGridSpec(
        num_scalar_prefetch=1,
        grid=(NH, NT),
        in_specs=[
            pl.BlockSpec((T, D_IN), lambda h, t, te: (t, 0)),
            pl.BlockSpec((1, D_IN, HC), lambda h, t, te: (te[t], 0, h)),
            pl.BlockSpec((1, 1, 1, HC), lambda h, t, te: (te[t], h, 0, 0)),
            pl.BlockSpec((1, HC, D_OUT), lambda h, t, te: (te[t], h, 0)),
            pl.BlockSpec((1, 1, D_OUT), lambda h, t, te: (te[t], 0, 0)),
        ],
        out_specs=pl.BlockSpec((1, T, D_OUT), lambda h, t, te: (h, t, 0)),
        scratch_shapes=[
            pltpu.VMEM((D_IN, HC), jnp.bfloat16),
            pltpu.VMEM((HC, D_OUT), jnp.bfloat16),
        ],
    )
    return pl.pallas_call(
        _mlp_body,
        grid_spec=grid_spec,
        out_shape=jax.ShapeDtypeStruct((NH, N_PAD, D_OUT), jnp.bfloat16),
        compiler_params=pltpu.CompilerParams(
            dimension_semantics=("arbitrary", "arbitrary"),
        ),
    )(tile_expert, x_pad, W1,
      b1.reshape(NUM_EXPERTS, NH, 1, HC), W2,
      b2.reshape(NUM_EXPERTS, 1, D_OUT))


def kernel(x, expert_indices, W1, b1, W2, b2):
    e = expert_indices.astype(jnp.int32)
    onehot = (e[:, None] == jnp.arange(NUM_EXPERTS, dtype=jnp.int32)[None, :])
    ranks_all = jnp.cumsum(onehot.astype(jnp.int32), axis=0)
    counts = ranks_all[-1]
    rank = jnp.take_along_axis(ranks_all, e[:, None], axis=1)[:, 0] - 1
    pad_counts = ((counts + T - 1) // T) * T
    pad_ends = jnp.cumsum(pad_counts)
    pad_starts = pad_ends - pad_counts
    dest = pad_starts[e] + rank
    g = jnp.zeros((N_PAD,), jnp.int32).at[dest].set(
        jnp.arange(N_TOK, dtype=jnp.int32))
    x_pad = x[g].astype(jnp.bfloat16)
    tile_expert = jnp.minimum(
        jnp.searchsorted(pad_ends, jnp.arange(NT, dtype=jnp.int32) * T,
                         side="right").astype(jnp.int32),
        NUM_EXPERTS - 1)
    y_parts = _grouped_mlp(x_pad, tile_expert, W1, b1, W2, b2)
    return y_parts.astype(jnp.float32).sum(axis=0)[dest]


# speedup vs baseline: 1.0614x; 1.0614x over previous
import functools

import jax
import jax.numpy as jnp
from jax.experimental import pallas as pl
from jax.experimental.pallas import tpu as pltpu

NUM_EXPERTS = 8
D_IN = 2048
D_HID = 8192
D_OUT = 2048
N_TOK = 8192

T = 256
HC = 1024
NH = D_HID // HC
N_PAD = N_TOK + NUM_EXPERTS * T
NT = N_PAD // T


def _mlp_body(te_ref, x_ref, w1_ref, b1_ref, w2_ref, b2_ref, o_ref):
    h = pl.program_id(0)
    xb = x_ref[...]
    w1 = w1_ref[0].astype(jnp.bfloat16)
    w2 = w2_ref[0].astype(jnp.bfloat16)
    hb = jnp.maximum(
        jnp.dot(xb, w1, preferred_element_type=jnp.float32) + b1_ref[0, 0],
        0.0).astype(jnp.bfloat16)
    part = jnp.dot(hb, w2, preferred_element_type=jnp.float32)

    @pl.when(h == 0)
    def _():
        o_ref[0] = (part + b2_ref[0]).astype(jnp.bfloat16)

    @pl.when(h != 0)
    def _():
        o_ref[0] = part.astype(jnp.bfloat16)


@functools.partial(jax.jit, static_argnames=())
def _grouped_mlp(x_pad, tile_expert, W1, b1, W2, b2):
    grid_spec = pltpu.PrefetchScalarGridSpec(
        num_scalar_prefetch=1,
        grid=(NH, NT),
        in_specs=[
            pl.BlockSpec((T, D_IN), lambda h, t, te: (t, 0)),
            pl.BlockSpec((1, D_IN, HC), lambda h, t, te: (te[t], 0, h)),
            pl.BlockSpec((1, 1, 1, HC), lambda h, t, te: (te[t], h, 0, 0)),
            pl.BlockSpec((1, HC, D_OUT), lambda h, t, te: (te[t], h, 0)),
            pl.BlockSpec((1, 1, D_OUT), lambda h, t, te: (te[t], 0, 0)),
        ],
        out_specs=pl.BlockSpec((1, T, D_OUT), lambda h, t, te: (h, t, 0)),
    )
    return pl.pallas_call(
        _mlp_body,
        grid_spec=grid_spec,
        out_shape=jax.ShapeDtypeStruct((NH, N_PAD, D_OUT), jnp.bfloat16),
        compiler_params=pltpu.CompilerParams(
            dimension_semantics=("arbitrary", "arbitrary"),
        ),
    )(tile_expert, x_pad, W1,
      b1.reshape(NUM_EXPERTS, NH, 1, HC), W2,
      b2.reshape(NUM_EXPERTS, 1, D_OUT))


def kernel(x, expert_indices, W1, b1, W2, b2):
    e = expert_indices.astype(jnp.int32)
    onehot = (e[:, None] == jnp.arange(NUM_EXPERTS, dtype=jnp.int32)[None, :])
    ranks_all = jnp.cumsum(onehot.astype(jnp.int32), axis=0)
    counts = ranks_all[-1]
    rank = jnp.take_along_axis(ranks_all, e[:, None], axis=1)[:, 0] - 1
    pad_counts = ((counts + T - 1) // T) * T
    pad_ends = jnp.cumsum(pad_counts)
    pad_starts = pad_ends - pad_counts
    dest = pad_starts[e] + rank
    g = jnp.zeros((N_PAD,), jnp.int32).at[dest].set(
        jnp.arange(N_TOK, dtype=jnp.int32))
    x_pad = x[g].astype(jnp.bfloat16)
    tile_expert = jnp.minimum(
        jnp.searchsorted(pad_ends, jnp.arange(NT, dtype=jnp.int32) * T,
                         side="right").astype(jnp.int32),
        NUM_EXPERTS - 1)
    y_parts = _grouped_mlp(x_pad, tile_expert, W1, b1, W2, b2)
    return y_parts.astype(jnp.float32).sum(axis=0)[dest]


# speedup vs baseline: 1.0815x; 1.0189x over previous
import functools

import jax
import jax.numpy as jnp
from jax.experimental import pallas as pl
from jax.experimental.pallas import tpu as pltpu

NUM_EXPERTS = 8
D_IN = 2048
D_HID = 8192
D_OUT = 2048
N_TOK = 8192

T = 256
HC = 1024
NH = D_HID // HC
N_PAD = N_TOK + NUM_EXPERTS * T
NT = N_PAD // T


def _mlp_body(te_ref, na_ref, x_ref, w1_ref, b1_ref, w2_ref, b2_ref, o_ref):
    h = pl.program_id(0)
    t = pl.program_id(1)

    @pl.when(t < na_ref[0])
    def _():
        xb = x_ref[...]
        w1 = w1_ref[0].astype(jnp.bfloat16)
        w2 = w2_ref[0].astype(jnp.bfloat16)
        hb = jnp.maximum(
            jnp.dot(xb, w1, preferred_element_type=jnp.float32) + b1_ref[0, 0],
            0.0).astype(jnp.bfloat16)
        part = jnp.dot(hb, w2, preferred_element_type=jnp.float32)

        @pl.when(h == 0)
        def _():
            o_ref[0] = (part + b2_ref[0]).astype(jnp.bfloat16)

        @pl.when(h != 0)
        def _():
            o_ref[0] = part.astype(jnp.bfloat16)


@functools.partial(jax.jit, static_argnames=())
def _grouped_mlp(x_pad, tile_expert, n_active, W1, b1, W2, b2):
    grid_spec = pltpu.PrefetchScalarGridSpec(
        num_scalar_prefetch=2,
        grid=(NH, NT),
        in_specs=[
            pl.BlockSpec((T, D_IN), lambda h, t, te, na: (t, 0)),
            pl.BlockSpec((1, D_IN, HC), lambda h, t, te, na: (te[t], 0, h)),
            pl.BlockSpec((1, 1, 1, HC), lambda h, t, te, na: (te[t], h, 0, 0)),
            pl.BlockSpec((1, HC, D_OUT), lambda h, t, te, na: (te[t], h, 0)),
            pl.BlockSpec((1, 1, D_OUT), lambda h, t, te, na: (te[t], 0, 0)),
        ],
        out_specs=pl.BlockSpec((1, T, D_OUT), lambda h, t, te, na: (h, t, 0)),
    )
    return pl.pallas_call(
        _mlp_body,
        grid_spec=grid_spec,
        out_shape=jax.ShapeDtypeStruct((NH, N_PAD, D_OUT), jnp.bfloat16),
        compiler_params=pltpu.CompilerParams(
            dimension_semantics=("arbitrary", "arbitrary"),
        ),
    )(tile_expert, n_active, x_pad, W1,
      b1.reshape(NUM_EXPERTS, NH, 1, HC), W2,
      b2.reshape(NUM_EXPERTS, 1, D_OUT))


def kernel(x, expert_indices, W1, b1, W2, b2):
    e = expert_indices.astype(jnp.int32)
    onehot = (e[:, None] == jnp.arange(NUM_EXPERTS, dtype=jnp.int32)[None, :])
    ranks_all = jnp.cumsum(onehot.astype(jnp.int32), axis=0)
    counts = ranks_all[-1]
    rank = jnp.take_along_axis(ranks_all, e[:, None], axis=1)[:, 0] - 1
    pad_counts = ((counts + T - 1) // T) * T
    pad_ends = jnp.cumsum(pad_counts)
    pad_starts = pad_ends - pad_counts
    dest = pad_starts[e] + rank
    g = jnp.zeros((N_PAD,), jnp.int32).at[dest].set(
        jnp.arange(N_TOK, dtype=jnp.int32))
    x_pad = x[g].astype(jnp.bfloat16)
    tile_expert = jnp.minimum(
        jnp.searchsorted(pad_ends, jnp.arange(NT, dtype=jnp.int32) * T,
                         side="right").astype(jnp.int32),
        NUM_EXPERTS - 1)
    n_active = (pad_ends[-1] // T).astype(jnp.int32).reshape(1)
    y_parts = _grouped_mlp(x_pad, tile_expert, n_active, W1, b1, W2, b2)
    return y_parts.astype(jnp.float32).sum(axis=0)[dest]


# speedup vs baseline: 1.2241x; 1.1319x over previous
import functools

import jax
import jax.numpy as jnp
from jax import lax
from jax.experimental import pallas as pl
from jax.experimental.pallas import tpu as pltpu
from jax.experimental.pallas import tpu_sc as plsc

NUM_EXPERTS = 8
D_IN = 2048
D_HID = 8192
D_OUT = 2048
N_TOK = 8192

T = 256
HC = 1024
NH = D_HID // HC
N_PAD = N_TOK + NUM_EXPERTS * T
NT = N_PAD // T


def _mlp_body(te_ref, na_ref, x_ref, w1_ref, b1_ref, w2_ref, b2_ref, o_ref):
    h = pl.program_id(0)
    t = pl.program_id(1)

    @pl.when(t < na_ref[0])
    def _():
        xb = x_ref[...]
        w1 = w1_ref[0].astype(jnp.bfloat16)
        w2 = w2_ref[0].astype(jnp.bfloat16)
        hb = jnp.maximum(
            jnp.dot(xb, w1, preferred_element_type=jnp.float32) + b1_ref[0, 0],
            0.0).astype(jnp.bfloat16)
        part = jnp.dot(hb, w2, preferred_element_type=jnp.float32)

        @pl.when(h == 0)
        def _():
            o_ref[0] = (part + b2_ref[0]).astype(jnp.bfloat16)

        @pl.when(h != 0)
        def _():
            o_ref[0] = part.astype(jnp.bfloat16)


@functools.partial(jax.jit, static_argnames=())
def _grouped_mlp(x_pad, tile_expert, n_active, W1, b1, W2, b2):
    grid_spec = pltpu.PrefetchScalarGridSpec(
        num_scalar_prefetch=2,
        grid=(NH, NT),
        in_specs=[
            pl.BlockSpec((T, D_IN), lambda h, t, te, na: (t, 0)),
            pl.BlockSpec((1, D_IN, HC), lambda h, t, te, na: (te[t], 0, h)),
            pl.BlockSpec((1, 1, 1, HC), lambda h, t, te, na: (te[t], h, 0, 0)),
            pl.BlockSpec((1, HC, D_OUT), lambda h, t, te, na: (te[t], h, 0)),
            pl.BlockSpec((1, 1, D_OUT), lambda h, t, te, na: (te[t], 0, 0)),
        ],
        out_specs=pl.BlockSpec((1, T, D_OUT), lambda h, t, te, na: (h, t, 0)),
    )
    return pl.pallas_call(
        _mlp_body,
        grid_spec=grid_spec,
        out_shape=jax.ShapeDtypeStruct((NH, N_PAD, D_OUT), jnp.bfloat16),
        compiler_params=pltpu.CompilerParams(
            dimension_semantics=("arbitrary", "arbitrary"),
        ),
    )(tile_expert, n_active, x_pad, W1,
      b1.reshape(NUM_EXPERTS, NH, 1, HC), W2,
      b2.reshape(NUM_EXPERTS, 1, D_OUT))


_NS = 16
_CHUNK = N_TOK // _NS
_NV = _CHUNK // 16
_NTP = 48


def _bcast(vec, lane):
    idx = jnp.full((16, 1), lane, dtype=jnp.int32)
    return lax.gather(
        vec, idx,
        dimension_numbers=lax.GatherDimensionNumbers(
            offset_dims=(), collapsed_slice_dims=(0,), start_index_map=(0,)),
        slice_sizes=(1,),
        mode=lax.GatherScatterMode.PROMISE_IN_BOUNDS)


def _sc_route_body(e_hbm, dest_out, te_out, na_out,
                   e_v, dest_v, te_v, na_v, all_cnt, shared_counts):
    c = lax.axis_index("c")
    s = lax.axis_index("s")

    @pl.when(c == 0)
    def _():
        lanes = lax.iota(jnp.int32, 16)
        one = jnp.full((16,), 1, jnp.int32)
        zero = jnp.zeros((16,), jnp.int32)
        base = s * _CHUNK
        pltpu.sync_copy(e_hbm.at[pl.ds(base, _CHUNK)], e_v)

        def _eq(a, b):
            return one - jnp.minimum(jnp.abs(a - b), one)

        c31 = jnp.full((16,), 31, jnp.int32)

        def _lt(a, b):
            return ((a - b) >> c31) & one

        def _prefix_incl(vec):
            acc = zero
            for k in range(16):
                kv = jnp.full((16,), k, jnp.int32)
                acc = acc + (one - _lt(lanes, kv)) * _bcast(vec, k)
            return acc

        cx = [zero] * NUM_EXPERTS
        for j in range(_NV):
            v = e_v[pl.ds(j * 16, 16)]
            for x in range(NUM_EXPERTS):
                xv = jnp.full((16,), x, jnp.int32)
                cx[x] = cx[x] + _eq(v, xv)
        ipx = [_prefix_incl(cx[x]) for x in range(NUM_EXPERTS)]
        counts = zero
        for x in range(NUM_EXPERTS):
            xv = jnp.full((16,), x, jnp.int32)
            counts = counts + _eq(lanes, xv) * _bcast(ipx[x], 15)

        na_v[...] = counts
        pltpu.sync_copy(na_v, shared_counts.at[pl.ds(s * 16, 16)])
        plsc.subcore_barrier()
        pltpu.sync_copy(shared_counts, all_cnt)
        total = zero
        prefix = zero
        s_vec = lax.broadcast(s, (16,)).astype(jnp.int32)
        for w in range(_NS):
            rv = all_cnt[pl.ds(w * 16, 16)]
            total = total + rv
            prefix = prefix + _lt(jnp.full((16,), w, jnp.int32), s_vec) * rv
        c255 = jnp.full((16,), T - 1, jnp.int32)
        c8 = jnp.full((16,), 8, jnp.int32)
        pad_counts = ((total + c255) >> c8) << c8
        pad_ends = _prefix_incl(pad_counts)
        seg_base = (pad_ends - pad_counts) + prefix

        running = [_bcast(seg_base, x) + (ipx[x] - cx[x])
                   for x in range(NUM_EXPERTS)]
        for j in range(_NV):
            v = e_v[pl.ds(j * 16, 16)]
            dest = zero
            for x in range(NUM_EXPERTS):
                xv = jnp.full((16,), x, jnp.int32)
                mi = _eq(v, xv)
                dest = dest + mi * (running[x] - dest)
                running[x] = running[x] + mi
            dest_v[pl.ds(j * 16, 16)] = dest
        pltpu.sync_copy(dest_v, dest_out.at[pl.ds(base, _CHUNK)])

        @pl.when(s == 0)
        def _():
            for j in range(_NTP // 16):
                r = (lanes + jnp.full((16,), 16 * j, jnp.int32)) \
                    * jnp.full((16,), T, jnp.int32)
                cnt = jnp.zeros((16,), jnp.int32)
                for x in range(NUM_EXPERTS):
                    cnt = cnt + (one - _lt(r, _bcast(pad_ends, x)))
                te_v[pl.ds(16 * j, 16)] = jnp.minimum(
                    cnt, jnp.full((16,), NUM_EXPERTS - 1, jnp.int32))
            na_v[...] = _bcast(pad_ends, NUM_EXPERTS - 1) >> c8
            pltpu.sync_copy(te_v.at[pl.ds(0, NT)], te_out)
            pltpu.sync_copy(na_v, na_out)


@jax.jit
def _sc_route(e):
    return pl.kernel(
        _sc_route_body,
        mesh=plsc.VectorSubcoreMesh(core_axis_name="c", subcore_axis_name="s"),
        out_type=[
            jax.ShapeDtypeStruct((N_TOK,), jnp.int32),
            jax.ShapeDtypeStruct((NT,), jnp.int32),
            jax.ShapeDtypeStruct((16,), jnp.int32),
        ],
        scratch_types=[
            pltpu.VMEM((_CHUNK,), jnp.int32),
            pltpu.VMEM((_CHUNK,), jnp.int32),
            pltpu.VMEM((_NTP,), jnp.int32),
            pltpu.VMEM((16,), jnp.int32),
            pltpu.VMEM((_NS * 16,), jnp.int32),
            pltpu.VMEM_SHARED((_NS * 16,), jnp.int32),
        ],
    )(e)


def kernel(x, expert_indices, W1, b1, W2, b2):
    e = expert_indices.astype(jnp.int32)
    dest, tile_expert, na = _sc_route(e)
    x_pad = jnp.zeros((N_PAD, D_IN), jnp.bfloat16).at[dest].set(
        x.astype(jnp.bfloat16))
    n_active = na[:1]
    y_parts = _grouped_mlp(x_pad, tile_expert, n_active, W1, b1, W2, b2)
    return y_parts.astype(jnp.float32).sum(axis=0)[dest]
